# baseline (speedup 1.0000x reference)
import functools

import jax
import jax.numpy as jnp
from jax import lax
from jax.experimental import pallas as pl
from jax.experimental.pallas import tpu as pltpu
from jax.experimental.pallas import tpu_sc as plsc

INPUT_SIZE = 128
MEMORY_SIZE = 100000
SPARSITY = 32
VALUE_SIZE = 64
LEARNING_RATE = 0.1
BATCH = 1024

M_PAD = 100352
NBKT = M_PAD // 128
NEG = -3.0e38
NW = 32
ROWS_PER_W = BATCH // NW


M_TILE = 1024
BPT = M_TILE // 128


def _matmul_body(keys_ref, proj_ref, scores_ref, bmax_ref):
    j = pl.program_id(0)
    s = jax.lax.dot_general(
        keys_ref[...], proj_ref[...],
        (((1,), (1,)), ((), ())),
        preferred_element_type=jnp.float32,
    )
    col = j * M_TILE + jax.lax.broadcasted_iota(jnp.int32, s.shape, 1)
    s = jnp.where(col < MEMORY_SIZE, s, NEG)
    s4 = s.reshape(BATCH // 8, 8, BPT, 128).transpose(0, 2, 1, 3)
    scores_ref[...] = s4.reshape(BATCH // 8, BPT * 8, 128)
    bmax_ref[...] = jnp.max(
        s.reshape(BATCH, BPT, 128), axis=2).transpose(1, 0)[:, :, None]


def _scores(keys, proj_pad):
    return pl.pallas_call(
        _matmul_body,
        grid=(M_PAD // M_TILE,),
        in_specs=[
            pl.BlockSpec((BATCH, INPUT_SIZE), lambda j: (0, 0)),
            pl.BlockSpec((M_TILE, INPUT_SIZE), lambda j: (j, 0)),
        ],
        out_specs=[
            pl.BlockSpec((BATCH // 8, BPT * 8, 128), lambda j: (0, j, 0)),
            pl.BlockSpec((BPT, BATCH, 1), lambda j: (j, 0, 0)),
        ],
        out_shape=[
            jax.ShapeDtypeStruct((BATCH // 8, NBKT * 8, 128), jnp.float32),
            jax.ShapeDtypeStruct((NBKT, BATCH, 1), jnp.float32),
        ],
    )(keys, proj_pad)


_IOTA = lambda: lax.iota(jnp.int32, 16)


def _sortkv(k, v):
    return plsc.sort_key_val(k, v, descending=True)


def _merge16(r0, r0i, r1, r1i, s, si):
    rs = lax.rev(s, (0,))
    rsi = lax.rev(si, (0,))
    t = r1 >= rs
    u = jnp.maximum(r1, rs)
    ui = jnp.where(t, r1i, rsi)
    u, ui = _sortkv(u, ui)
    ru = lax.rev(u, (0,))
    rui = lax.rev(ui, (0,))
    t0 = r0 >= ru
    n0 = jnp.maximum(r0, ru)
    n0i = jnp.where(t0, r0i, rui)
    n1 = jnp.minimum(r0, ru)
    n1i = jnp.where(t0, rui, r0i)
    n0, n0i = _sortkv(n0, n0i)
    n1, n1i = _sortkv(n1, n1i)
    return n0, n0i, n1, n1i


def _merge_if_useful(carry, v, vi):
    r0, r0i, r1, r1i, runmin = carry
    mx = lax.reduce_max(v, axes=(0,))

    def do(r0, r0i, r1, r1i, _rm):
        s, si = _sortkv(v, vi)
        n0, n0i, n1, n1i = _merge16(r0, r0i, r1, r1i, s, si)
        return n0, n0i, n1, n1i, lax.reduce_min(n1, axes=(0,))

    def skip(r0, r0i, r1, r1i, rm):
        return r0, r0i, r1, r1i, rm

    return lax.cond(mx > runmin, do, skip, r0, r0i, r1, r1i, runmin)


def _init_top32(v0, v0i, v1, v1i):
    a, ai = _sortkv(v0, v0i)
    b, bi = _sortkv(v1, v1i)
    rb = lax.rev(b, (0,))
    rbi = lax.rev(bi, (0,))
    t = a >= rb
    hi = jnp.maximum(a, rb)
    hii = jnp.where(t, ai, rbi)
    lo = jnp.minimum(a, rb)
    loi = jnp.where(t, rbi, ai)
    r0, r0i = _sortkv(hi, hii)
    r1, r1i = _sortkv(lo, loi)
    return r0, r0i, r1, r1i, lax.reduce_min(r1, axes=(0,))


def _extract_at(vals, j16):
    return lax.reduce_max(
        jnp.where(_IOTA() == j16, vals, NEG if vals.dtype == jnp.float32
                  else jnp.int32(-2147483647)), axes=(0,))


def _k23_body(bmax_hbm, scores_hbm, out_hbm, bmax_v, idx_c, cand_c, idx_d,
              cand_d, out_v, sem_c, sem_d):
    wid = lax.axis_index("s") * 2 + lax.axis_index("c")
    iota = _IOTA()
    base_r = wid * ROWS_PER_W

    def s1(rr):
        pltpu.sync_copy(bmax_hbm.at[base_r + rr], bmax_v)
        v0 = bmax_v[pl.ds(0, 16)]
        v1 = bmax_v[pl.ds(16, 16)]
        st = _init_top32(v0, iota, v1, iota + 16)
        for j in (2, 3, 48):
            st = _merge_if_useful(st, bmax_v[pl.ds(j * 16, 16)],
                                  iota + j * 16)

        def s1_body(g, st):
            j = 4 + g * 4
            vs = [bmax_v[pl.ds((j + t) * 16, 16)] for t in range(4)]
            gm = jnp.maximum(jnp.maximum(vs[0], vs[1]),
                             jnp.maximum(vs[2], vs[3]))
            mx = lax.reduce_max(gm, axes=(0,))

            def do(st):
                for t in range(4):
                    st = _merge_if_useful(st, vs[t], iota + (j + t) * 16)
                return st

            return lax.cond(mx > st[4], do, lambda s: s, st)

        return lax.fori_loop(0, 11, s1_body, st, unroll=2)

    def fire(st, rr, idx_ref, cand_ref, sem):
        r = base_r + rr
        base = (r // 8) * (NBKT * 8) + (r % 8)
        idx_ref[pl.ds(0, 16)] = st[1] * 8 + base
        idx_ref[pl.ds(16, 16)] = st[3] * 8 + base
        pltpu.async_copy(scores_hbm.at[idx_ref], cand_ref, sem)

    def waitg(idx_ref, cand_ref, sem):
        pltpu.make_async_copy(scores_hbm.at[idx_ref], cand_ref, sem).wait()

    def s2(st, cand_v, rr):
        r0, r0i, r1, r1i, _ = st
        bid0 = _extract_at(r0i, 0)
        c0 = cand_v[0, pl.ds(0, 16)]
        c1 = cand_v[0, pl.ds(16, 16)]
        cst = _init_top32(c0, bid0 * 128 + iota, c1, bid0 * 128 + 16 + iota)
        for k in range(2, 8):
            cst = _merge_if_useful(
                cst, cand_v[0, pl.ds(k * 16, 16)], bid0 * 128 + k * 16 + iota)

        def s2_body(j, cst):
            j16 = jnp.where(j < 16, j, j - 16)
            bmax_j = _extract_at(jnp.where(j < 16, r0, r1), j16)
            bid_j = _extract_at(jnp.where(j < 16, r0i, r1i), j16)

            def do_bucket(cst):
                stt = cst
                for k in range(8):
                    stt = _merge_if_useful(
                        stt, cand_v[j, pl.ds(k * 16, 16)],
                        bid_j * 128 + k * 16 + iota)
                return stt

            return lax.cond(bmax_j > cst[4], do_bucket, lambda c: c, cst)

        cst = lax.fori_loop(1, SPARSITY, s2_body, cst)
        out_v[pl.ds(0, 16)] = cst[1]
        out_v[pl.ds(16, 16)] = cst[3]
        pltpu.sync_copy(out_v, out_hbm.at[base_r + rr])

    st0 = s1(0)
    fire(st0, 0, idx_c, cand_c, sem_c)

    def body(i, st_even):
        st_odd = s1(2 * i + 1)
        fire(st_odd, 2 * i + 1, idx_d, cand_d, sem_d)
        waitg(idx_c, cand_c, sem_c)
        s2(st_even, cand_c, 2 * i)
        st_even2 = s1(2 * i + 2)
        fire(st_even2, 2 * i + 2, idx_c, cand_c, sem_c)
        waitg(idx_d, cand_d, sem_d)
        s2(st_odd, cand_d, 2 * i + 1)
        return st_even2

    st30 = lax.fori_loop(0, ROWS_PER_W // 2 - 1, body, st0)
    st31 = s1(ROWS_PER_W - 1)
    fire(st31, ROWS_PER_W - 1, idx_d, cand_d, sem_d)
    waitg(idx_c, cand_c, sem_c)
    s2(st30, cand_c, ROWS_PER_W - 2)
    waitg(idx_d, cand_d, sem_d)
    s2(st31, cand_d, ROWS_PER_W - 1)


def _topk_sc(bmax, scores2d):
    mesh = plsc.VectorSubcoreMesh(core_axis_name="c", subcore_axis_name="s")
    return pl.kernel(
        _k23_body,
        mesh=mesh,
        compiler_params=pltpu.CompilerParams(needs_layout_passes=False),
        out_type=jax.ShapeDtypeStruct((BATCH, SPARSITY), jnp.int32),
        scratch_types=[
            pltpu.VMEM((NBKT,), jnp.float32),
            pltpu.VMEM((SPARSITY,), jnp.int32),
            pltpu.VMEM((SPARSITY, 128), jnp.float32),
            pltpu.VMEM((SPARSITY,), jnp.int32),
            pltpu.VMEM((SPARSITY, 128), jnp.float32),
            pltpu.VMEM((SPARSITY,), jnp.int32),
            pltpu.SemaphoreType.DMA,
            pltpu.SemaphoreType.DMA,
        ],
    )(bmax, scores2d)


HALF_M = MEMORY_SIZE // 2
PAD_ROWS = 176
HV = VALUE_SIZE // 2
BPW = BATCH // NW
PAIRS = BPW * SPARSITY


def _k4_body(idx_hbm, tgt_hbm, mem_hbm, memg_hbm, out_hbm, shared, idx_v,
             uidx_v, chunk_v, gat_v, dl_v, sem):
    c = lax.axis_index("c")
    s = lax.axis_index("s")
    iota = _IOTA()
    scale = LEARNING_RATE / SPARSITY

    pltpu.sync_copy(idx_hbm.at[pl.ds(s * 16, 16)], idx_v)

    def grp_body(q, _):
        pltpu.async_copy(memg_hbm.at[idx_v.at[q]], gat_v, sem).wait()

        def b_body(b, _):
            accs = tuple(jnp.zeros((16,), jnp.float32) for _ in range(4))

            def s_body(k, accs):
                row = b * SPARSITY + k
                return tuple(a + gat_v[row, pl.ds(h * 16, 16)]
                             for h, a in enumerate(accs))

            accs = lax.fori_loop(0, SPARSITY, s_body, accs)
            for h in range(4):
                dl_v[q * 4 + b, pl.ds(h * 16, 16)] = accs[h]
            return 0

        lax.fori_loop(0, 4, b_body, 0)
        return 0

    lax.fori_loop(0, 16, grp_body, 0)

    pltpu.sync_copy(tgt_hbm.at[pl.ds(s * 64, 64)], chunk_v.at[pl.ds(0, 64)])

    def d_body(b, _):
        for h in range(4):
            t = chunk_v[b, pl.ds(h * 16, 16)]
            r = dl_v[b, pl.ds(h * 16, 16)]
            dl_v[b, pl.ds(h * 16, 16)] = (t - r) * scale
        return 0

    lax.fori_loop(0, 64, d_body, 0)

    for lo_rel, w in ((0, 12504), (12504, 12504),
                      (25008, 12504), (37512, 12488)):
        lo = c * HALF_M + lo_rel
        rem = w - 15 * 784

        @pl.when(s < 15)
        def _load():
            pltpu.sync_copy(mem_hbm.at[pl.ds(lo + s * 784, 784)],
                            shared.at[pl.ds(s * 784, 784)])

        @pl.when(s == 15)
        def _load_last():
            pltpu.sync_copy(mem_hbm.at[pl.ds(lo + 15 * 784, rem)],
                            shared.at[pl.ds(15 * 784, rem)])

        plsc.subcore_barrier()

        def m_body(q, _):
            for e in range(8):
                v = idx_v[q, pl.ds(e * 16, 16)]
                ok = (v >= lo) & (v < lo + w)
                uidx_v[q, pl.ds(e * 16, 16)] = jnp.where(
                    ok, v - lo, w + s * 8 + (iota % 8))
            return 0

        lax.fori_loop(0, 16, m_body, 0)

        def sc_body(q, _):
            def bld(b4, _):
                d = [dl_v[q * 4 + b4, pl.ds(h * 16, 16)] for h in range(4)]

                def rep(k, _):
                    for h in range(4):
                        chunk_v[b4 * SPARSITY + k, pl.ds(h * 16, 16)] = d[h]
                    return 0

                lax.fori_loop(0, SPARSITY, rep, 0)
                return 0

            lax.fori_loop(0, 4, bld, 0)
            pltpu.sync_copy(chunk_v, shared.at[uidx_v.at[q]], add=True)
            return 0

        lax.fori_loop(0, 16, sc_body, 0)
        plsc.subcore_barrier()

        @pl.when(s < 15)
        def _store():
            pltpu.sync_copy(shared.at[pl.ds(s * 784, 784)],
                            out_hbm.at[pl.ds(lo + s * 784, 784)])

        @pl.when(s == 15)
        def _store_last():
            pltpu.sync_copy(shared.at[pl.ds(15 * 784, rem)],
                            out_hbm.at[pl.ds(lo + 15 * 784, rem)])

        plsc.subcore_barrier()


def _tail_sc(indices, targets, mem_value):
    mesh = plsc.VectorSubcoreMesh(core_axis_name="c", subcore_axis_name="s")
    idx2 = indices.reshape(BATCH * SPARSITY // 128, 128)
    return pl.kernel(
        _k4_body,
        mesh=mesh,
        compiler_params=pltpu.CompilerParams(needs_layout_passes=False),
        out_type=jax.ShapeDtypeStruct((MEMORY_SIZE, VALUE_SIZE), jnp.float32),
        scratch_types=[
            pltpu.VMEM_SHARED((12504 + PAD_ROWS, VALUE_SIZE), jnp.float32),
            pltpu.VMEM((16, 128), jnp.int32),
            pltpu.VMEM((16, 128), jnp.int32),
            pltpu.VMEM((128, VALUE_SIZE), jnp.float32),
            pltpu.VMEM((128, 2 * VALUE_SIZE), jnp.float32),
            pltpu.VMEM((64, VALUE_SIZE), jnp.float32),
            pltpu.SemaphoreType.DMA,
        ],
    )(idx2, targets, mem_value,
      jnp.pad(mem_value, ((0, 0), (0, VALUE_SIZE))))


def kernel(keys, targets, proj, mem_value):
    proj_pad = jnp.pad(proj, ((0, M_PAD - MEMORY_SIZE), (0, 0)))
    scores3, bmax3 = _scores(keys, proj_pad)
    scores2d = scores3.reshape(BATCH // 8 * NBKT * 8, 128)
    bmax = bmax3.reshape(NBKT, BATCH).T
    indices = _topk_sc(bmax, scores2d)
    return _tail_sc(indices, targets, mem_value)


# speedup vs baseline: 1.0077x; 1.0077x over previous
import jax
import jax.numpy as jnp
from jax import lax
from jax.experimental import pallas as pl
from jax.experimental.pallas import tpu as pltpu
from jax.experimental.pallas import tpu_sc as plsc

INPUT_SIZE = 128
MEMORY_SIZE = 100000
SPARSITY = 32
VALUE_SIZE = 64
LEARNING_RATE = 0.1
BATCH = 1024

M_PAD = 100352
NBKT = M_PAD // 128
NEG = -3.0e38
NW = 32
ROWS_PER_W = BATCH // NW


M_TILE = 1024
BPT = M_TILE // 128


def _matmul_body(keys_ref, proj_ref, scores_ref, bmax_ref):
    j = pl.program_id(0)
    s = jax.lax.dot_general(
        keys_ref[...], proj_ref[...],
        (((1,), (1,)), ((), ())),
        preferred_element_type=jnp.float32,
    )
    col = j * M_TILE + jax.lax.broadcasted_iota(jnp.int32, s.shape, 1)
    s = jnp.where(col < MEMORY_SIZE, s, NEG)
    s4 = s.reshape(BATCH // 8, 8, BPT, 128).transpose(0, 2, 1, 3)
    scores_ref[...] = s4.reshape(BATCH // 8, BPT * 8, 128)
    bmax_ref[...] = jnp.max(
        s.reshape(BATCH, BPT, 128), axis=2).transpose(1, 0)[:, :, None]


def _scores(keys, proj_pad):
    return pl.pallas_call(
        _matmul_body,
        grid=(M_PAD // M_TILE,),
        in_specs=[
            pl.BlockSpec((BATCH, INPUT_SIZE), lambda j: (0, 0)),
            pl.BlockSpec((M_TILE, INPUT_SIZE), lambda j: (j, 0)),
        ],
        out_specs=[
            pl.BlockSpec((BATCH // 8, BPT * 8, 128), lambda j: (0, j, 0)),
            pl.BlockSpec((BPT, BATCH, 1), lambda j: (j, 0, 0)),
        ],
        out_shape=[
            jax.ShapeDtypeStruct((BATCH // 8, NBKT * 8, 128), jnp.float32),
            jax.ShapeDtypeStruct((NBKT, BATCH, 1), jnp.float32),
        ],
    )(keys, proj_pad)


_IOTA = lambda: lax.iota(jnp.int32, 16)


def _sortkv(k, v):
    return plsc.sort_key_val(k, v, descending=True)


def _merge16(r0, r0i, r1, r1i, s, si):
    rs = lax.rev(s, (0,))
    rsi = lax.rev(si, (0,))
    t = r1 >= rs
    u = jnp.maximum(r1, rs)
    ui = jnp.where(t, r1i, rsi)
    u, ui = _sortkv(u, ui)
    ru = lax.rev(u, (0,))
    rui = lax.rev(ui, (0,))
    t0 = r0 >= ru
    n0 = jnp.maximum(r0, ru)
    n0i = jnp.where(t0, r0i, rui)
    n1 = jnp.minimum(r0, ru)
    n1i = jnp.where(t0, rui, r0i)
    n0, n0i = _sortkv(n0, n0i)
    n1, n1i = _sortkv(n1, n1i)
    return n0, n0i, n1, n1i


def _merge_if_useful(carry, v, vi):
    r0, r0i, r1, r1i, runmin = carry
    mx = lax.reduce_max(v, axes=(0,))

    def do(r0, r0i, r1, r1i, _rm):
        s, si = _sortkv(v, vi)
        n0, n0i, n1, n1i = _merge16(r0, r0i, r1, r1i, s, si)
        return n0, n0i, n1, n1i, lax.reduce_min(n1, axes=(0,))

    def skip(r0, r0i, r1, r1i, rm):
        return r0, r0i, r1, r1i, rm

    return lax.cond(mx > runmin, do, skip, r0, r0i, r1, r1i, runmin)


def _init_top32(v0, v0i, v1, v1i):
    a, ai = _sortkv(v0, v0i)
    b, bi = _sortkv(v1, v1i)
    rb = lax.rev(b, (0,))
    rbi = lax.rev(bi, (0,))
    t = a >= rb
    hi = jnp.maximum(a, rb)
    hii = jnp.where(t, ai, rbi)
    lo = jnp.minimum(a, rb)
    loi = jnp.where(t, rbi, ai)
    r0, r0i = _sortkv(hi, hii)
    r1, r1i = _sortkv(lo, loi)
    return r0, r0i, r1, r1i, lax.reduce_min(r1, axes=(0,))


def _extract_at(vals, j16):
    return lax.reduce_max(
        jnp.where(_IOTA() == j16, vals, NEG if vals.dtype == jnp.float32
                  else jnp.int32(-2147483647)), axes=(0,))


def _k23_body(bmax_hbm, scores_hbm, out_hbm, bmax_v, idx_c, cand_c, idx_d,
              cand_d, out_v, sem_c, sem_d):
    wid = lax.axis_index("s") * 2 + lax.axis_index("c")
    iota = _IOTA()
    base_r = wid * ROWS_PER_W

    def s1(rr):
        pltpu.sync_copy(bmax_hbm.at[base_r + rr], bmax_v)
        v0 = bmax_v[pl.ds(0, 16)]
        v1 = bmax_v[pl.ds(16, 16)]
        st = _init_top32(v0, iota, v1, iota + 16)

        def s1_body(j, st):
            v = bmax_v[pl.ds(j * 16, 16)]
            return _merge_if_useful(st, v, iota + j * 16)

        return lax.fori_loop(2, NBKT // 16, s1_body, st, unroll=4)

    def fire(st, rr, idx_ref, cand_ref, sem):
        r = base_r + rr
        base = (r // 8) * (NBKT * 8) + (r % 8)
        idx_ref[pl.ds(0, 16)] = st[1] * 8 + base
        idx_ref[pl.ds(16, 16)] = st[3] * 8 + base
        pltpu.async_copy(scores_hbm.at[idx_ref], cand_ref, sem)

    def waitg(idx_ref, cand_ref, sem):
        pltpu.make_async_copy(scores_hbm.at[idx_ref], cand_ref, sem).wait()

    def s2(st, cand_v, rr):
        r0, r0i, r1, r1i, _ = st
        bid0 = _extract_at(r0i, 0)
        c0 = cand_v[0, pl.ds(0, 16)]
        c1 = cand_v[0, pl.ds(16, 16)]
        cst = _init_top32(c0, bid0 * 128 + iota, c1, bid0 * 128 + 16 + iota)
        for k in range(2, 8):
            cst = _merge_if_useful(
                cst, cand_v[0, pl.ds(k * 16, 16)], bid0 * 128 + k * 16 + iota)

        def s2_body(j, cst):
            j16 = jnp.where(j < 16, j, j - 16)
            bmax_j = _extract_at(jnp.where(j < 16, r0, r1), j16)
            bid_j = _extract_at(jnp.where(j < 16, r0i, r1i), j16)

            def do_bucket(cst):
                stt = cst
                for k in range(8):
                    stt = _merge_if_useful(
                        stt, cand_v[j, pl.ds(k * 16, 16)],
                        bid_j * 128 + k * 16 + iota)
                return stt

            return lax.cond(bmax_j > cst[4], do_bucket, lambda c: c, cst)

        cst = lax.fori_loop(1, SPARSITY, s2_body, cst)
        out_v[pl.ds(0, 16)] = cst[1]
        out_v[pl.ds(16, 16)] = cst[3]
        pltpu.sync_copy(out_v, out_hbm.at[base_r + rr])

    st0 = s1(0)
    fire(st0, 0, idx_c, cand_c, sem_c)

    def body(i, st_even):
        st_odd = s1(2 * i + 1)
        fire(st_odd, 2 * i + 1, idx_d, cand_d, sem_d)
        waitg(idx_c, cand_c, sem_c)
        s2(st_even, cand_c, 2 * i)
        st_even2 = s1(2 * i + 2)
        fire(st_even2, 2 * i + 2, idx_c, cand_c, sem_c)
        waitg(idx_d, cand_d, sem_d)
        s2(st_odd, cand_d, 2 * i + 1)
        return st_even2

    st30 = lax.fori_loop(0, ROWS_PER_W // 2 - 1, body, st0)
    st31 = s1(ROWS_PER_W - 1)
    fire(st31, ROWS_PER_W - 1, idx_d, cand_d, sem_d)
    waitg(idx_c, cand_c, sem_c)
    s2(st30, cand_c, ROWS_PER_W - 2)
    waitg(idx_d, cand_d, sem_d)
    s2(st31, cand_d, ROWS_PER_W - 1)


def _topk_sc(bmax, scores2d):
    mesh = plsc.VectorSubcoreMesh(core_axis_name="c", subcore_axis_name="s")
    return pl.kernel(
        _k23_body,
        mesh=mesh,
        compiler_params=pltpu.CompilerParams(needs_layout_passes=False),
        out_type=jax.ShapeDtypeStruct((BATCH, SPARSITY), jnp.int32),
        scratch_types=[
            pltpu.VMEM((NBKT,), jnp.float32),
            pltpu.VMEM((SPARSITY,), jnp.int32),
            pltpu.VMEM((SPARSITY, 128), jnp.float32),
            pltpu.VMEM((SPARSITY,), jnp.int32),
            pltpu.VMEM((SPARSITY, 128), jnp.float32),
            pltpu.VMEM((SPARSITY,), jnp.int32),
            pltpu.SemaphoreType.DMA,
            pltpu.SemaphoreType.DMA,
        ],
    )(bmax, scores2d)


HALF_M = MEMORY_SIZE // 2
PAD_ROWS = 176
HV = VALUE_SIZE // 2
BPW = BATCH // NW
PAIRS = BPW * SPARSITY


def _k4_body(idx_hbm, tgt_hbm, mem_hbm, memg_hbm, out_hbm, shared, idx_v,
             uidx_v, chunk_v, gat_v, dl_v, sem):
    c = lax.axis_index("c")
    s = lax.axis_index("s")
    iota = _IOTA()
    scale = LEARNING_RATE / SPARSITY

    pltpu.sync_copy(idx_hbm.at[pl.ds(s * 16, 16)], idx_v)

    def grp_body(q, _):
        pltpu.async_copy(memg_hbm.at[idx_v.at[q]], gat_v, sem).wait()

        def b_body(b, _):
            accs = tuple(jnp.zeros((16,), jnp.float32) for _ in range(4))

            def s_body(k, accs):
                row = b * SPARSITY + k
                return tuple(a + gat_v[row, pl.ds(h * 16, 16)]
                             for h, a in enumerate(accs))

            accs = lax.fori_loop(0, SPARSITY, s_body, accs)
            for h in range(4):
                dl_v[q * 4 + b, pl.ds(h * 16, 16)] = accs[h]
            return 0

        lax.fori_loop(0, 4, b_body, 0)
        return 0

    lax.fori_loop(0, 16, grp_body, 0)

    pltpu.sync_copy(tgt_hbm.at[pl.ds(s * 64, 64)], chunk_v.at[pl.ds(0, 64)])

    def d_body(b, _):
        for h in range(4):
            t = chunk_v[b, pl.ds(h * 16, 16)]
            r = dl_v[b, pl.ds(h * 16, 16)]
            dl_v[b, pl.ds(h * 16, 16)] = (t - r) * scale
        return 0

    lax.fori_loop(0, 64, d_body, 0)

    for lo_rel, w in ((0, 12504), (12504, 12504),
                      (25008, 12504), (37512, 12488)):
        lo = c * HALF_M + lo_rel
        rem = w - 15 * 784

        @pl.when(s < 15)
        def _load():
            pltpu.sync_copy(mem_hbm.at[pl.ds(lo + s * 784, 784)],
                            shared.at[pl.ds(s * 784, 784)])

        @pl.when(s == 15)
        def _load_last():
            pltpu.sync_copy(mem_hbm.at[pl.ds(lo + 15 * 784, rem)],
                            shared.at[pl.ds(15 * 784, rem)])

        plsc.subcore_barrier()

        def m_body(q, _):
            for e in range(8):
                v = idx_v[q, pl.ds(e * 16, 16)]
                ok = (v >= lo) & (v < lo + w)
                uidx_v[q, pl.ds(e * 16, 16)] = jnp.where(
                    ok, v - lo, w + s * 8 + (iota % 8))
            return 0

        lax.fori_loop(0, 16, m_body, 0)

        def sc_body(q, _):
            def bld(b4, _):
                d = [dl_v[q * 4 + b4, pl.ds(h * 16, 16)] for h in range(4)]

                def rep(k, _):
                    for h in range(4):
                        chunk_v[b4 * SPARSITY + k, pl.ds(h * 16, 16)] = d[h]
                    return 0

                lax.fori_loop(0, SPARSITY, rep, 0)
                return 0

            lax.fori_loop(0, 4, bld, 0)
            pltpu.sync_copy(chunk_v, shared.at[uidx_v.at[q]], add=True)
            return 0

        lax.fori_loop(0, 16, sc_body, 0)
        plsc.subcore_barrier()

        @pl.when(s < 15)
        def _store():
            pltpu.sync_copy(shared.at[pl.ds(s * 784, 784)],
                            out_hbm.at[pl.ds(lo + s * 784, 784)])

        @pl.when(s == 15)
        def _store_last():
            pltpu.sync_copy(shared.at[pl.ds(15 * 784, rem)],
                            out_hbm.at[pl.ds(lo + 15 * 784, rem)])

        plsc.subcore_barrier()


def _tail_sc(indices, targets, mem_value):
    mesh = plsc.VectorSubcoreMesh(core_axis_name="c", subcore_axis_name="s")
    idx2 = indices.reshape(BATCH * SPARSITY // 128, 128)
    return pl.kernel(
        _k4_body,
        mesh=mesh,
        compiler_params=pltpu.CompilerParams(needs_layout_passes=False),
        out_type=jax.ShapeDtypeStruct((MEMORY_SIZE, VALUE_SIZE), jnp.float32),
        scratch_types=[
            pltpu.VMEM_SHARED((12504 + PAD_ROWS, VALUE_SIZE), jnp.float32),
            pltpu.VMEM((16, 128), jnp.int32),
            pltpu.VMEM((16, 128), jnp.int32),
            pltpu.VMEM((128, VALUE_SIZE), jnp.float32),
            pltpu.VMEM((128, 2 * VALUE_SIZE), jnp.float32),
            pltpu.VMEM((64, VALUE_SIZE), jnp.float32),
            pltpu.SemaphoreType.DMA,
        ],
    )(idx2, targets, mem_value,
      jnp.pad(mem_value, ((0, 0), (0, VALUE_SIZE))))


def kernel(keys, targets, proj, mem_value):
    proj_pad = jnp.pad(proj, ((0, M_PAD - MEMORY_SIZE), (0, 0)))
    scores3, bmax3 = _scores(keys, proj_pad)
    scores2d = scores3.reshape(BATCH // 8 * NBKT * 8, 128)
    bmax = bmax3.reshape(NBKT, BATCH).T
    indices = _topk_sc(bmax, scores2d)
    return _tail_sc(indices, targets, mem_value)
